# Initial kernel scaffold; baseline (speedup 1.0000x reference)
import jax
import jax.numpy as jnp
from jax.experimental import pallas as pl


def kernel(H, adj, W):
    raise NotImplementedError("write your pallas kernel here")


# speedup vs baseline: 1.2961x; 1.2961x over previous
import jax
import jax.numpy as jnp
from jax.experimental import pallas as pl
from jax.experimental.pallas import tpu as pltpu


def _round_up(x, m):
    return (x + m - 1) // m * m


def _prep_kernel(adj_ref, h_ref, wt_ref, adj_i8_ref, d_ref, xs_ref, acc_ref):
    k = pl.program_id(1)

    a = adj_ref[...]
    adj_i8_ref[...] = a.astype(jnp.int8)

    @pl.when(k == 0)
    def _():
        acc_ref[...] = a

    @pl.when(k > 0)
    def _():
        acc_ref[...] += a

    @pl.when(k == pl.num_programs(1) - 1)
    def _():
        deg = jnp.sum(acc_ref[...], axis=1, keepdims=True)
        d = jax.lax.rsqrt(deg + 1.0)
        d_ref[...] = d
        x = jnp.dot(h_ref[...], wt_ref[...],
                    preferred_element_type=jnp.float32)
        xs_ref[...] = (x * d).astype(xs_ref.dtype)


def _make_spmm_kernel(tm):
    def _spmm_kernel(d_ref, adj_ref, xs_ref, o_ref):
        i = pl.program_id(0)
        row0 = pl.multiple_of(i * tm, 128)
        acc = jnp.dot(adj_ref[...].astype(jnp.bfloat16), xs_ref[...],
                      preferred_element_type=jnp.float32)
        acc += xs_ref[pl.ds(row0, tm), :].astype(jnp.float32)
        o_ref[...] = jnp.maximum(acc * d_ref[...], 0.0)

    return _spmm_kernel


def kernel(H, adj, W):
    N, F_in = H.shape
    F_out = W.shape[0]

    n_pad = _round_up(N, 128)
    fi_pad = _round_up(F_in, 128)
    fo_pad = _round_up(F_out, 128)
    tm = tk = 512
    while n_pad % tm:
        tm -= 128
        tk = tm

    h_p = jnp.pad(H.astype(jnp.float32), ((0, n_pad - N), (0, fi_pad - F_in)))
    wt_p = jnp.pad(W.astype(jnp.float32).T,
                   ((0, fi_pad - F_in), (0, fo_pad - F_out)))
    adj_p = jnp.pad(adj.astype(jnp.float32),
                    ((0, n_pad - N), (0, n_pad - N)))

    grid_rows = n_pad // tm
    grid_red = n_pad // tk

    vmem_limit = 64 * 1024 * 1024
    params_2d = pltpu.CompilerParams(
        dimension_semantics=("parallel", "arbitrary"),
        vmem_limit_bytes=vmem_limit)
    params_1d = pltpu.CompilerParams(
        dimension_semantics=("parallel",),
        vmem_limit_bytes=vmem_limit)

    adj_i8, d, xs = pl.pallas_call(
        _prep_kernel,
        out_shape=(jax.ShapeDtypeStruct((n_pad, n_pad), jnp.int8),
                   jax.ShapeDtypeStruct((n_pad, 1), jnp.float32),
                   jax.ShapeDtypeStruct((n_pad, fo_pad), jnp.bfloat16)),
        grid_spec=pltpu.PrefetchScalarGridSpec(
            num_scalar_prefetch=0,
            grid=(grid_rows, grid_red),
            in_specs=[
                pl.BlockSpec((tm, tk), lambda i, k: (i, k)),
                pl.BlockSpec((tm, fi_pad), lambda i, k: (i, 0)),
                pl.BlockSpec((fi_pad, fo_pad), lambda i, k: (0, 0)),
            ],
            out_specs=[
                pl.BlockSpec((tm, tk), lambda i, k: (i, k)),
                pl.BlockSpec((tm, 1), lambda i, k: (i, 0)),
                pl.BlockSpec((tm, fo_pad), lambda i, k: (i, 0)),
            ],
            scratch_shapes=[pltpu.VMEM((tm, tk), jnp.float32)]),
        compiler_params=params_2d,
    )(adj_p, h_p, wt_p)

    cost = pl.CostEstimate(
        flops=2 * n_pad * n_pad * fo_pad + 3 * n_pad * fo_pad,
        transcendentals=0,
        bytes_accessed=(n_pad * n_pad + n_pad * fo_pad * 2
                        + n_pad * fo_pad * 4 + n_pad * 4))

    out_p = pl.pallas_call(
        _make_spmm_kernel(tm),
        out_shape=jax.ShapeDtypeStruct((n_pad, fo_pad), jnp.float32),
        grid_spec=pltpu.PrefetchScalarGridSpec(
            num_scalar_prefetch=0,
            grid=(grid_rows,),
            in_specs=[
                pl.BlockSpec((tm, 1), lambda i: (i, 0)),
                pl.BlockSpec((tm, n_pad), lambda i: (i, 0)),
                pl.BlockSpec((n_pad, fo_pad), lambda i: (0, 0)),
            ],
            out_specs=pl.BlockSpec((tm, fo_pad), lambda i: (i, 0))),
        compiler_params=params_1d,
        cost_estimate=cost,
    )(d, adj_i8, xs)

    return out_p[:N, :F_out]


# speedup vs baseline: 1.3512x; 1.0425x over previous
import jax
import jax.numpy as jnp
from jax.experimental import pallas as pl
from jax.experimental.pallas import tpu as pltpu

_F8 = jnp.float8_e4m3fn
_LO_SCALE = 128.0


def _round_up(x, m):
    return (x + m - 1) // m * m


def _make_kernel(tm, tk, n_pad, fo_pad):
    def _body(adj_ref, h_ref, wt_ref, o_ref,
              adj8_ref, xs8_ref, d_ref, acc_ref):
        p = pl.program_id(0)
        i = pl.program_id(1)
        k = pl.program_id(2)
        row0 = pl.multiple_of(i * tm, 128)
        col0 = pl.multiple_of(k * tk, 128)

        @pl.when(p == 0)
        def _phase0():
            a = adj_ref[...]
            adj8_ref[pl.ds(row0, tm), pl.ds(col0, tk)] = a.astype(_F8)

            @pl.when(k == 0)
            def _():
                acc_ref[...] = a

            @pl.when(k > 0)
            def _():
                acc_ref[...] += a

            @pl.when(k == pl.num_programs(2) - 1)
            def _():
                deg = jnp.sum(acc_ref[...], axis=1, keepdims=True)
                d = jax.lax.rsqrt(deg + 1.0)
                d_ref[pl.ds(row0, tm), :] = d
                x = jnp.dot(h_ref[...], wt_ref[...],
                            preferred_element_type=jnp.float32) * d
                hi = x.astype(_F8)
                xs8_ref[pl.ds(row0, tm), :fo_pad] = hi
                xs8_ref[pl.ds(row0, tm), fo_pad:] = (
                    (x - hi.astype(jnp.float32)) * _LO_SCALE).astype(_F8)

        @pl.when((p == 1) & (k == 0))
        def _phase1():
            wide = jnp.dot(adj8_ref[pl.ds(row0, tm), :], xs8_ref[...],
                           preferred_element_type=jnp.float32)
            xs8_row = xs8_ref[pl.ds(row0, tm), :].astype(jnp.float32)
            acc = (wide[:, :fo_pad] + xs8_row[:, :fo_pad]
                   + (wide[:, fo_pad:] + xs8_row[:, fo_pad:])
                   * (1.0 / _LO_SCALE))
            o_ref[...] = jnp.maximum(acc * d_ref[pl.ds(row0, tm), :], 0.0)

    return _body


def kernel(H, adj, W):
    N, F_in = H.shape
    F_out = W.shape[0]

    n_pad = _round_up(N, 128)
    fi_pad = _round_up(F_in, 128)
    fo_pad = _round_up(F_out, 128)
    tm = tk = 512
    while n_pad % tm:
        tm -= 128
        tk = tm

    h_p = jnp.pad(H.astype(jnp.float32), ((0, n_pad - N), (0, fi_pad - F_in)))
    wt_p = jnp.pad(W.astype(jnp.float32).T,
                   ((0, fi_pad - F_in), (0, fo_pad - F_out)))
    adj_p = jnp.pad(adj.astype(jnp.float32),
                    ((0, n_pad - N), (0, n_pad - N)))

    grid_rows = n_pad // tm
    grid_red = n_pad // tk

    out_p = pl.pallas_call(
        _make_kernel(tm, tk, n_pad, fo_pad),
        out_shape=jax.ShapeDtypeStruct((n_pad, fo_pad), jnp.float32),
        grid_spec=pltpu.PrefetchScalarGridSpec(
            num_scalar_prefetch=0,
            grid=(2, grid_rows, grid_red),
            in_specs=[
                pl.BlockSpec((tm, tk),
                             lambda p, i, k: ((1 - p) * i, (1 - p) * k)),
                pl.BlockSpec((tm, fi_pad), lambda p, i, k: ((1 - p) * i, 0)),
                pl.BlockSpec((fi_pad, fo_pad), lambda p, i, k: (0, 0)),
            ],
            out_specs=pl.BlockSpec((tm, fo_pad), lambda p, i, k: (p * i, 0)),
            scratch_shapes=[
                pltpu.VMEM((n_pad, n_pad), _F8),
                pltpu.VMEM((n_pad, 2 * fo_pad), _F8),
                pltpu.VMEM((n_pad, 1), jnp.float32),
                pltpu.VMEM((tm, tk), jnp.float32),
            ]),
        compiler_params=pltpu.CompilerParams(
            dimension_semantics=("arbitrary", "arbitrary", "arbitrary"),
            vmem_limit_bytes=60 * 1024 * 1024),
    )(adj_p, h_p, wt_p)

    return out_p[:N, :F_out]


# speedup vs baseline: 1.3531x; 1.0014x over previous
import jax
import jax.numpy as jnp
from jax.experimental import pallas as pl
from jax.experimental.pallas import tpu as pltpu

_F8 = jnp.float8_e4m3fn
_LO_SCALE = 128.0


def _round_up(x, m):
    return (x + m - 1) // m * m


def _make_kernel(tm, tk, n_pad, fo_pad):
    def _body(adj_ref, h_ref, wt_ref, o_ref,
              adj8_ref, xs8_ref, d_ref, acc_ref):
        p = pl.program_id(0)
        i = pl.program_id(1)
        k = pl.program_id(2)
        row0 = pl.multiple_of(i * tm, 128)
        col0 = pl.multiple_of(k * tk, 128)

        @pl.when(p == 0)
        def _phase0():
            a = adj_ref[...]
            adj8_ref[pl.ds(row0, tm), pl.ds(col0, tk)] = a.astype(_F8)

            @pl.when(i * tm == k * tk)
            def _():
                rids = jax.lax.broadcasted_iota(jnp.int32, (tm, tk), 0)
                cids = jax.lax.broadcasted_iota(jnp.int32, (tm, tk), 1)
                eye = jnp.where(rids == cids, 1.0, 0.0).astype(jnp.float32)
                adj8_ref[pl.ds(row0, tm), pl.ds(col0, tk)] = (
                    a + eye).astype(_F8)

            @pl.when(k == 0)
            def _():
                acc_ref[...] = a

            @pl.when(k > 0)
            def _():
                acc_ref[...] += a

            @pl.when(k == pl.num_programs(2) - 1)
            def _():
                deg = jnp.sum(acc_ref[...], axis=1, keepdims=True)
                d = jax.lax.rsqrt(deg + 1.0)
                d_ref[pl.ds(row0, tm), :] = d
                x = jnp.dot(h_ref[...], wt_ref[...],
                            preferred_element_type=jnp.float32) * d
                hi = x.astype(_F8)
                xs8_ref[pl.ds(row0, tm), :fo_pad] = hi
                xs8_ref[pl.ds(row0, tm), fo_pad:] = (
                    (x - hi.astype(jnp.float32)) * _LO_SCALE).astype(_F8)

        @pl.when((p == 1) & (k == 0))
        def _phase1():
            wide = jnp.dot(adj8_ref[pl.ds(row0, tm), :], xs8_ref[...],
                           preferred_element_type=jnp.float32)
            acc = wide[:, :fo_pad] + wide[:, fo_pad:] * (1.0 / _LO_SCALE)
            o_ref[...] = jnp.maximum(acc * d_ref[pl.ds(row0, tm), :], 0.0)

    return _body


def kernel(H, adj, W):
    N, F_in = H.shape
    F_out = W.shape[0]

    n_pad = _round_up(N, 128)
    fi_pad = _round_up(F_in, 128)
    fo_pad = _round_up(F_out, 128)
    tm = tk = 512
    while n_pad % tm:
        tm -= 128
        tk = tm

    h_p = jnp.pad(H.astype(jnp.float32), ((0, n_pad - N), (0, fi_pad - F_in)))
    wt_p = jnp.pad(W.astype(jnp.float32).T,
                   ((0, fi_pad - F_in), (0, fo_pad - F_out)))
    adj_p = jnp.pad(adj.astype(jnp.float32),
                    ((0, n_pad - N), (0, n_pad - N)))

    grid_rows = n_pad // tm
    grid_red = n_pad // tk

    out_p = pl.pallas_call(
        _make_kernel(tm, tk, n_pad, fo_pad),
        out_shape=jax.ShapeDtypeStruct((n_pad, fo_pad), jnp.float32),
        grid_spec=pltpu.PrefetchScalarGridSpec(
            num_scalar_prefetch=0,
            grid=(2, grid_rows, grid_red),
            in_specs=[
                pl.BlockSpec((tm, tk),
                             lambda p, i, k: ((1 - p) * i, (1 - p) * k)),
                pl.BlockSpec((tm, fi_pad), lambda p, i, k: ((1 - p) * i, 0)),
                pl.BlockSpec((fi_pad, fo_pad), lambda p, i, k: (0, 0)),
            ],
            out_specs=pl.BlockSpec((tm, fo_pad), lambda p, i, k: (p * i, 0)),
            scratch_shapes=[
                pltpu.VMEM((n_pad, n_pad), _F8),
                pltpu.VMEM((n_pad, 2 * fo_pad), _F8),
                pltpu.VMEM((n_pad, 1), jnp.float32),
                pltpu.VMEM((tm, tk), jnp.float32),
            ]),
        compiler_params=pltpu.CompilerParams(
            dimension_semantics=("arbitrary", "arbitrary", "arbitrary"),
            vmem_limit_bytes=60 * 1024 * 1024),
    )(adj_p, h_p, wt_p)

    return out_p[:N, :F_out]


# speedup vs baseline: 2.1779x; 1.6095x over previous
import jax
import jax.numpy as jnp
from jax.experimental import pallas as pl
from jax.experimental.pallas import tpu as pltpu

_F8 = jnp.float8_e4m3fn
_LO_SCALE = 128.0


def _round_up(x, m):
    return (x + m - 1) // m * m


def _make_kernel(tm, n_pad, fo_pad):
    def _body(adj_ref, h_ref, wt_ref, o_ref, adj8_ref, xs8_ref, d_ref):
        p = pl.program_id(0)
        i = pl.program_id(1)
        row0 = pl.multiple_of(i * tm, 128)

        @pl.when(p == 0)
        def _phase0():
            band = adj_ref[...]
            adj8_ref[pl.ds(row0, tm), :] = band.astype(_F8)

            rids = jax.lax.broadcasted_iota(jnp.int32, (tm, tm), 0)
            cids = jax.lax.broadcasted_iota(jnp.int32, (tm, tm), 1)
            eye = jnp.where(rids == cids, 1.0, 0.0).astype(jnp.float32)
            adj8_ref[pl.ds(row0, tm), pl.ds(row0, tm)] = (
                adj_ref[:, pl.ds(row0, tm)] + eye).astype(_F8)

            deg = jnp.sum(band, axis=1, keepdims=True)
            d = jax.lax.rsqrt(deg + 1.0)
            d_ref[pl.ds(row0, tm), :] = d
            x = jnp.dot(h_ref[...], wt_ref[...],
                        preferred_element_type=jnp.float32) * d
            hi = x.astype(_F8)
            xs8_ref[pl.ds(row0, tm), :fo_pad] = hi
            xs8_ref[pl.ds(row0, tm), fo_pad:] = (
                (x - hi.astype(jnp.float32)) * _LO_SCALE).astype(_F8)

        @pl.when(p == 1)
        def _phase1():
            wide = jnp.dot(adj8_ref[pl.ds(row0, tm), :], xs8_ref[...],
                           preferred_element_type=jnp.float32)
            acc = wide[:, :fo_pad] + wide[:, fo_pad:] * (1.0 / _LO_SCALE)
            o_ref[...] = jnp.maximum(acc * d_ref[pl.ds(row0, tm), :], 0.0)

    return _body


def kernel(H, adj, W):
    N, F_in = H.shape
    F_out = W.shape[0]

    n_pad = _round_up(N, 128)
    fi_pad = _round_up(F_in, 128)
    fo_pad = _round_up(F_out, 128)
    tm = 256
    while n_pad % tm:
        tm -= 128

    h_p = jnp.pad(H.astype(jnp.float32), ((0, n_pad - N), (0, fi_pad - F_in)))
    wt_p = jnp.pad(W.astype(jnp.float32).T,
                   ((0, fi_pad - F_in), (0, fo_pad - F_out)))
    adj_p = jnp.pad(adj.astype(jnp.float32),
                    ((0, n_pad - N), (0, n_pad - N)))

    grid_rows = n_pad // tm

    out_p = pl.pallas_call(
        _make_kernel(tm, n_pad, fo_pad),
        out_shape=jax.ShapeDtypeStruct((n_pad, fo_pad), jnp.float32),
        grid_spec=pltpu.PrefetchScalarGridSpec(
            num_scalar_prefetch=0,
            grid=(2, grid_rows),
            in_specs=[
                pl.BlockSpec((tm, n_pad), lambda p, i: ((1 - p) * i, 0)),
                pl.BlockSpec((tm, fi_pad), lambda p, i: ((1 - p) * i, 0)),
                pl.BlockSpec((fi_pad, fo_pad), lambda p, i: (0, 0)),
            ],
            out_specs=pl.BlockSpec((tm, fo_pad), lambda p, i: (p * i, 0)),
            scratch_shapes=[
                pltpu.VMEM((n_pad, n_pad), _F8),
                pltpu.VMEM((n_pad, 2 * fo_pad), _F8),
                pltpu.VMEM((n_pad, 1), jnp.float32),
            ]),
        compiler_params=pltpu.CompilerParams(
            dimension_semantics=("arbitrary", "arbitrary"),
            vmem_limit_bytes=60 * 1024 * 1024),
    )(adj_p, h_p, wt_p)

    return out_p[:N, :F_out]


# speedup vs baseline: 2.2473x; 1.0319x over previous
import jax
import jax.numpy as jnp
from jax.experimental import pallas as pl
from jax.experimental.pallas import tpu as pltpu

_F8 = jnp.float8_e4m3fn
_LO_SCALE = 128.0


def _round_up(x, m):
    return (x + m - 1) // m * m


def _make_kernel(tm, n_pad, fo_pad):
    def _body(adj_ref, h_ref, wt_ref, o_ref,
              band8_ref, xs8_ref, d_ref, acc_ref):
        k = pl.program_id(0)
        row0 = pl.multiple_of(k * tm, 128)

        band = adj_ref[...]
        band8_ref[...] = band.astype(_F8)

        rids = jax.lax.broadcasted_iota(jnp.int32, (tm, tm), 0)
        cids = jax.lax.broadcasted_iota(jnp.int32, (tm, tm), 1)
        eye = jnp.where(rids == cids, 1.0, 0.0).astype(jnp.float32)
        band8_ref[:, pl.ds(row0, tm)] = (
            adj_ref[:, pl.ds(row0, tm)] + eye).astype(_F8)

        deg = jnp.sum(band, axis=1, keepdims=True)
        d = jax.lax.rsqrt(deg + 1.0)
        d_ref[pl.ds(row0, tm), :] = d
        x = jnp.dot(h_ref[...], wt_ref[...],
                    preferred_element_type=jnp.float32) * d
        hi = x.astype(_F8)
        xs8_ref[:, :fo_pad] = hi
        xs8_ref[:, fo_pad:] = ((x - hi.astype(jnp.float32))
                               * _LO_SCALE).astype(_F8)

        part = jax.lax.dot_general(
            band8_ref[...], xs8_ref[...],
            dimension_numbers=(((0,), (0,)), ((), ())),
            preferred_element_type=jnp.float32)
        comb = part[:, :fo_pad] + part[:, fo_pad:] * (1.0 / _LO_SCALE)

        @pl.when(k == 0)
        def _():
            acc_ref[...] = comb

        @pl.when(k > 0)
        def _():
            acc_ref[...] += comb

        @pl.when(k == pl.num_programs(0) - 1)
        def _():
            o_ref[...] = jnp.maximum(acc_ref[...] * d_ref[...], 0.0)

    return _body


def kernel(H, adj, W):
    N, F_in = H.shape
    F_out = W.shape[0]

    n_pad = _round_up(N, 128)
    fi_pad = _round_up(F_in, 128)
    fo_pad = _round_up(F_out, 128)
    tm = 512
    while n_pad % tm:
        tm -= 128

    h_p = jnp.pad(H.astype(jnp.float32), ((0, n_pad - N), (0, fi_pad - F_in)))
    wt_p = jnp.pad(W.astype(jnp.float32).T,
                   ((0, fi_pad - F_in), (0, fo_pad - F_out)))
    adj_p = jnp.pad(adj.astype(jnp.float32),
                    ((0, n_pad - N), (0, n_pad - N)))

    grid_rows = n_pad // tm

    out_p = pl.pallas_call(
        _make_kernel(tm, n_pad, fo_pad),
        out_shape=jax.ShapeDtypeStruct((n_pad, fo_pad), jnp.float32),
        grid_spec=pltpu.PrefetchScalarGridSpec(
            num_scalar_prefetch=0,
            grid=(grid_rows,),
            in_specs=[
                pl.BlockSpec((tm, n_pad), lambda k: (k, 0)),
                pl.BlockSpec((tm, fi_pad), lambda k: (k, 0)),
                pl.BlockSpec((fi_pad, fo_pad), lambda k: (0, 0)),
            ],
            out_specs=pl.BlockSpec((n_pad, fo_pad), lambda k: (0, 0)),
            scratch_shapes=[
                pltpu.VMEM((tm, n_pad), _F8),
                pltpu.VMEM((tm, 2 * fo_pad), _F8),
                pltpu.VMEM((n_pad, 1), jnp.float32),
                pltpu.VMEM((n_pad, fo_pad), jnp.float32),
            ]),
        compiler_params=pltpu.CompilerParams(
            dimension_semantics=("arbitrary",),
            vmem_limit_bytes=60 * 1024 * 1024),
    )(adj_p, h_p, wt_p)

    return out_p[:N, :F_out]


# speedup vs baseline: 2.8374x; 1.2626x over previous
import jax
import jax.numpy as jnp
from jax.experimental import pallas as pl
from jax.experimental.pallas import tpu as pltpu


def _round_up(x, m):
    return (x + m - 1) // m * m


def _make_kernel(tm, n_pad, fo_pad):
    def _body(adj_ref, h_ref, wt_ref, o_ref, xs_ref, d_ref):
        k = pl.program_id(0)
        row0 = pl.multiple_of(k * tm, 128)

        band = adj_ref[...]
        deg = jnp.sum(band, axis=1, keepdims=True)
        d = jax.lax.rsqrt(deg + 1.0)
        d_ref[pl.ds(row0, tm), :] = d
        x = jnp.dot(h_ref[...], wt_ref[...],
                    preferred_element_type=jnp.float32) * d
        xs_ref[...] = x

        part = jax.lax.dot_general(
            band, xs_ref[...],
            dimension_numbers=(((0,), (0,)), ((), ())),
            preferred_element_type=jnp.float32)

        @pl.when(k == 0)
        def _():
            o_ref[...] = part

        @pl.when(k > 0)
        def _():
            o_ref[...] += part

        o_ref[pl.ds(row0, tm), :] += xs_ref[...]

        @pl.when(k == pl.num_programs(0) - 1)
        def _():
            o_ref[...] = jnp.maximum(o_ref[...] * d_ref[...], 0.0)

    return _body


def kernel(H, adj, W):
    N, F_in = H.shape
    F_out = W.shape[0]

    n_pad = _round_up(N, 128)
    fi_pad = _round_up(F_in, 128)
    fo_pad = _round_up(F_out, 128)
    tm = 1024
    while n_pad % tm:
        tm -= 128

    h_p = jnp.pad(H.astype(jnp.float32), ((0, n_pad - N), (0, fi_pad - F_in)))
    wt_p = jnp.pad(W.astype(jnp.float32).T,
                   ((0, fi_pad - F_in), (0, fo_pad - F_out)))
    adj_p = jnp.pad(adj.astype(jnp.float32),
                    ((0, n_pad - N), (0, n_pad - N)))

    grid_rows = n_pad // tm

    out_p = pl.pallas_call(
        _make_kernel(tm, n_pad, fo_pad),
        out_shape=jax.ShapeDtypeStruct((n_pad, fo_pad), jnp.float32),
        grid_spec=pltpu.PrefetchScalarGridSpec(
            num_scalar_prefetch=0,
            grid=(grid_rows,),
            in_specs=[
                pl.BlockSpec((tm, n_pad), lambda k: (k, 0)),
                pl.BlockSpec((tm, fi_pad), lambda k: (k, 0)),
                pl.BlockSpec((fi_pad, fo_pad), lambda k: (0, 0)),
            ],
            out_specs=pl.BlockSpec((n_pad, fo_pad), lambda k: (0, 0)),
            scratch_shapes=[
                pltpu.VMEM((tm, fo_pad), jnp.float32),
                pltpu.VMEM((n_pad, 1), jnp.float32),
            ]),
        compiler_params=pltpu.CompilerParams(
            dimension_semantics=("arbitrary",),
            vmem_limit_bytes=60 * 1024 * 1024),
    )(adj_p, h_p, wt_p)

    return out_p[:N, :F_out]


# speedup vs baseline: 2.9728x; 1.0477x over previous
import jax
import jax.numpy as jnp
from jax.experimental import pallas as pl
from jax.experimental.pallas import tpu as pltpu


def _round_up(x, m):
    return (x + m - 1) // m * m


def _make_kernel(tm, n_pad, fo_pad):
    def _body(adj_ref, h_ref, wt_ref, o_ref, xs_ref, d_ref):
        k = pl.program_id(0)
        row0 = pl.multiple_of(k * tm, 128)

        band = adj_ref[...]
        deg = jnp.sum(band, axis=1, keepdims=True)
        d = jax.lax.rsqrt(deg + 1.0)
        d_ref[pl.ds(row0, tm), :] = d
        x = jax.lax.dot_general(
            h_ref[...], wt_ref[...],
            dimension_numbers=(((1,), (1,)), ((), ())),
            preferred_element_type=jnp.float32) * d
        xs_ref[...] = x

        part = jax.lax.dot_general(
            band, xs_ref[...],
            dimension_numbers=(((0,), (0,)), ((), ())),
            preferred_element_type=jnp.float32)

        @pl.when(k == 0)
        def _():
            o_ref[...] = part

        @pl.when(k > 0)
        def _():
            o_ref[...] += part

        o_ref[pl.ds(row0, tm), :] += xs_ref[...]

        @pl.when(k == pl.num_programs(0) - 1)
        def _():
            o_ref[...] = jnp.maximum(o_ref[...] * d_ref[...], 0.0)

    return _body


def kernel(H, adj, W):
    N, F_in = H.shape
    F_out = W.shape[0]

    n_pad = _round_up(N, 128)
    fi_pad = _round_up(F_in, 128)
    fo_pad = _round_up(F_out, 128)
    tm = 1024
    while n_pad % tm:
        tm -= 128

    h_p = jnp.pad(H.astype(jnp.float32), ((0, n_pad - N), (0, fi_pad - F_in)))
    w_p = jnp.pad(W.astype(jnp.float32),
                  ((0, fo_pad - F_out), (0, fi_pad - F_in)))
    adj_p = jnp.pad(adj.astype(jnp.float32),
                    ((0, n_pad - N), (0, n_pad - N)))

    grid_rows = n_pad // tm

    out_p = pl.pallas_call(
        _make_kernel(tm, n_pad, fo_pad),
        out_shape=jax.ShapeDtypeStruct((n_pad, fo_pad), jnp.float32),
        grid_spec=pltpu.PrefetchScalarGridSpec(
            num_scalar_prefetch=0,
            grid=(grid_rows,),
            in_specs=[
                pl.BlockSpec((tm, n_pad), lambda k: (k, 0)),
                pl.BlockSpec((tm, fi_pad), lambda k: (k, 0)),
                pl.BlockSpec((fo_pad, fi_pad), lambda k: (0, 0)),
            ],
            out_specs=pl.BlockSpec((n_pad, fo_pad), lambda k: (0, 0)),
            scratch_shapes=[
                pltpu.VMEM((tm, fo_pad), jnp.float32),
                pltpu.VMEM((n_pad, 1), jnp.float32),
            ]),
        compiler_params=pltpu.CompilerParams(
            dimension_semantics=("arbitrary",),
            vmem_limit_bytes=60 * 1024 * 1024),
    )(adj_p, h_p, w_p)

    return out_p[:N, :F_out]
